# Initial kernel scaffold; baseline (speedup 1.0000x reference)
import jax
import jax.numpy as jnp
from jax.experimental import pallas as pl


def kernel(x, edge_index, ln_gamma, ln_beta, weights, bias):
    raise NotImplementedError("write your pallas kernel here")


# speedup vs baseline: 12.3926x; 12.3926x over previous
import functools
import math

import jax
import jax.numpy as jnp
from jax import lax
from jax.experimental import pallas as pl
from jax.experimental.pallas import tpu as pltpu
from jax.experimental.pallas import tpu_sc as plsc

NB = 8
D = 128
DA = 144
NC = 2
NS = 16
NW = NC * NS
CHUNK = 80


def _sc_aggregate(xa, src, dst):
    n = xa.shape[0]
    e = src.shape[0]
    per_tile = e // NW
    n_chunks = per_tile // CHUNK
    rows_per_sub = n // NS

    mesh = plsc.VectorSubcoreMesh(
        core_axis_name="c", subcore_axis_name="s", num_cores=NC, num_subcores=NS
    )

    @functools.partial(
        pl.kernel,
        out_type=jax.ShapeDtypeStruct((NC, n, DA), jnp.float32),
        mesh=mesh,
        scratch_types=[
            pltpu.VMEM((CHUNK,), jnp.int32),
            pltpu.VMEM((CHUNK,), jnp.int32),
            pltpu.VMEM((CHUNK, DA), jnp.float32),
            pltpu.VMEM_SHARED((n, DA), jnp.float32),
            pltpu.SemaphoreType.DMA,
        ],
        compiler_params=pltpu.CompilerParams(use_tc_tiling_on_sc=False),
    )
    def sc_agg(xa_hbm, src_hbm, dst_hbm, out_hbm, dst_v, src_v, rows_v, acc_sh, sem):
        c = lax.axis_index("c")
        s = lax.axis_index("s")
        tile = c * NS + s
        row0 = s * rows_per_sub
        pltpu.sync_copy(
            xa_hbm.at[pl.ds(row0, rows_per_sub)], acc_sh.at[pl.ds(row0, rows_per_sub)]
        )
        plsc.subcore_barrier()

        base0 = tile * per_tile

        def body(i, carry):
            base = base0 + i * CHUNK
            pltpu.sync_copy(dst_hbm.at[pl.ds(base, CHUNK)], dst_v)
            pltpu.async_copy(xa_hbm.at[dst_v], rows_v, sem).wait()
            pltpu.sync_copy(src_hbm.at[pl.ds(base, CHUNK)], src_v)
            pltpu.sync_copy(rows_v, acc_sh.at[src_v], add=True)
            return carry

        lax.fori_loop(0, n_chunks, body, 0)
        plsc.subcore_barrier()
        pltpu.sync_copy(
            acc_sh.at[pl.ds(row0, rows_per_sub)],
            out_hbm.at[c, pl.ds(row0, rows_per_sub)],
        )

    return sc_agg(xa, src, dst)


def _tc_transform(s_pair, x, ln_gamma, ln_beta, weights, bias, blk):
    n = x.shape[0]
    centers = [-1.0 + 2.0 * f / (NB - 1) for f in range(NB)]
    sigma = (2.0 / (NB - 1)) / 2.0
    inv_denom = 1.0 / (2.0 * sigma * sigma)

    def body(s0_ref, s1_ref, x_ref, g_ref, b_ref, w_ref, bias_ref, out_ref):
        s0 = s0_ref[...]
        s1 = s1_ref[...]
        xx = x_ref[...]
        agg_sum = s0[:, :D] + s1[:, :D] - xx
        deg = s0[:, D : D + 1] + s1[:, D : D + 1] - 1.0
        agg = agg_sum / deg
        mu = jnp.mean(agg, axis=1, keepdims=True)
        cen = agg - mu
        var = jnp.mean(cen * cen, axis=1, keepdims=True)
        h = cen * lax.rsqrt(var + 1e-5) * g_ref[...] + b_ref[...]
        acc = jnp.zeros((blk, D), jnp.float32)
        for f in range(NB):
            dh = h - centers[f]
            basis = jnp.exp(dh * dh * (-inv_denom))
            acc = acc + jnp.dot(
                basis, w_ref[:, f, :], preferred_element_type=jnp.float32
            )
        out_ref[...] = acc + bias_ref[...]

    grid = n // blk
    return pl.pallas_call(
        body,
        grid=(grid,),
        in_specs=[
            pl.BlockSpec((blk, DA), lambda i: (i, 0)),
            pl.BlockSpec((blk, DA), lambda i: (i, 0)),
            pl.BlockSpec((blk, D), lambda i: (i, 0)),
            pl.BlockSpec((1, D), lambda i: (0, 0)),
            pl.BlockSpec((1, D), lambda i: (0, 0)),
            pl.BlockSpec((D, NB, D), lambda i: (0, 0, 0)),
            pl.BlockSpec((1, D), lambda i: (0, 0)),
        ],
        out_specs=pl.BlockSpec((blk, D), lambda i: (i, 0)),
        out_shape=jax.ShapeDtypeStruct((n, D), jnp.float32),
    )(
        s_pair[0],
        s_pair[1],
        x,
        ln_gamma.reshape(1, D),
        ln_beta.reshape(1, D),
        weights,
        bias.reshape(1, D),
    )


def kernel(x, edge_index, ln_gamma, ln_beta, weights, bias):
    n = x.shape[0]
    src = edge_index[0].astype(jnp.int32)
    dst = edge_index[1].astype(jnp.int32)
    ones_col = jnp.ones((n, 1), jnp.float32)
    pad = jnp.zeros((n, DA - D - 1), jnp.float32)
    xa = jnp.concatenate([x, ones_col, pad], axis=1)
    s_pair = _sc_aggregate(xa, src, dst)
    return _tc_transform(s_pair, x, ln_gamma, ln_beta, weights, bias, blk=1000)
